# baseline (speedup 1.0000x reference)
import functools
import math

import jax
import jax.numpy as jnp
from jax import lax
from jax.experimental import pallas as pl
from jax.experimental.pallas import tpu as pltpu
from jax.experimental.pallas import tpu_sc as plsc

EMB = 64
ROW = 128
LANES = 16
CHUNK = 128
NBUF = 4
AHEAD = 2
OBUF = 2


@functools.lru_cache(maxsize=None)
def _build(t_dim: int, b_dim: int, n_words: int):
    n_rows = t_dim * b_dim
    info = plsc.get_sparse_core_info()
    nc, ns = info.num_cores, info.num_subcores
    nw = nc * ns
    rpw = n_rows // nw
    chunks = rpw // CHUNK
    assert rpw % CHUNK == 0 and chunks % NBUF == 0
    outer = chunks // NBUF
    bb_per_t = b_dim // CHUNK

    mesh = plsc.VectorSubcoreMesh(core_axis_name="c", subcore_axis_name="s")

    @functools.partial(
        pl.kernel,
        out_type=jax.ShapeDtypeStruct((n_rows * EMB,), jnp.float32),
        mesh=mesh,
        scratch_types=[
            pltpu.VMEM((chunks, CHUNK), jnp.int32),
            pltpu.VMEM((NBUF, CHUNK, ROW), jnp.float32),
            pltpu.VMEM((EMB * CHUNK,), jnp.float32),
            pltpu.VMEM((EMB * CHUNK,), jnp.float32),
            pltpu.VMEM((EMB,), jnp.float32),
            pltpu.SemaphoreType.DMA((NBUF,)),
            pltpu.SemaphoreType.DMA((OBUF,)),
        ],
        compiler_params=pltpu.CompilerParams(needs_layout_passes=False),
    )
    def emb_kernel(tok_hbm, table_hbm, pos_hbm, out_hbm, idx_all, gbufs,
                   obuf0, obuf1, pos_v, gsem, osem):
        obufs = [obuf0, obuf1]
        wid = lax.axis_index("s") * nc + lax.axis_index("c")
        base_chunk = wid * chunks

        pltpu.sync_copy(pos_hbm, pos_v)
        p = [pos_v[pl.ds(q * LANES, LANES)] for q in range(EMB // LANES)]
        pltpu.sync_copy(tok_hbm.at[pl.ds(base_chunk, chunks)], idx_all)
        lane_off = lax.iota(jnp.int32, LANES) * CHUNK

        def gather(c, b):
            pltpu.async_copy(table_hbm.at[idx_all.at[c]], gbufs.at[b],
                             gsem.at[b])

        def gather_wait(c, b):
            pltpu.make_async_copy(table_hbm.at[idx_all.at[c]], gbufs.at[b],
                                  gsem.at[b]).wait()

        def stores(c, m):
            g = base_chunk + c
            t = g // bb_per_t
            bb = g % bb_per_t
            blk = 8 * CHUNK
            for cb in range(EMB // 8):
                off = ((t * (EMB // 8) + cb) * bb_per_t + bb) * blk
                pltpu.async_copy(
                    obufs[m].at[pl.ds(cb * blk, blk)],
                    out_hbm.at[pl.ds(off, blk)], osem.at[m])

        def stores_wait(m):
            blk = 8 * CHUNK
            for _ in range(EMB // 8):
                pltpu.make_async_copy(
                    obufs[m].at[pl.ds(0, blk)],
                    out_hbm.at[pl.ds(0, blk)], osem.at[m]).wait()

        for f in range(AHEAD):
            gather(f, f)

        def outer_body(o, carry):
            for b in range(NBUF):
                c = o * NBUF + b
                f = c + AHEAD
                m = b % OBUF

                fb = (b + AHEAD) % NBUF

                @pl.when(f < chunks)
                def _issue_gather():
                    gather(f, fb)

                @pl.when(c > OBUF - 1)
                def _wait_stores():
                    stores_wait(m)

                gather_wait(c, b)

                def row_body(bl, cc):
                    for q in range(EMB // LANES):
                        v = gbufs[b, bl, pl.ds(q * LANES, LANES)] + p[q]
                        plsc.store_scatter(
                            obufs[m], [lane_off + (q * LANES * CHUNK + bl)],
                            v)
                    return cc

                lax.fori_loop(0, CHUNK, row_body, 0)
                stores(c, m)
            return carry

        lax.fori_loop(0, outer, outer_body, 0)

        for k in range(chunks - OBUF, chunks):
            stores_wait(k % OBUF)

    return emb_kernel


def kernel(tokens, table):
    t_dim, b_dim = tokens.shape
    n_rows = t_dim * b_dim
    n_words, emb = table.shape

    den = jnp.exp(-jnp.arange(0, emb, 2, dtype=jnp.float32) * math.log(10000.0) / emb)
    pos = jnp.zeros((emb,), dtype=jnp.float32)
    pos = pos.at[0::2].set(jnp.sin(t_dim * den))
    pos = pos.at[1::2].set(jnp.cos(t_dim * den))

    tok = tokens.reshape(n_rows // CHUNK, CHUNK).astype(jnp.int32)
    table_p = jnp.pad(table, ((0, 0), (0, ROW - emb)))
    out = _build(t_dim, b_dim, n_words)(tok, table_p, pos)
    out5 = out.reshape(t_dim, emb // 8, b_dim // CHUNK, 8, CHUNK)

    return out5.transpose(0, 2, 4, 1, 3).reshape(t_dim, b_dim, emb)


# speedup vs baseline: 1.7234x; 1.7234x over previous
import functools
import math

import jax
import jax.numpy as jnp
from jax import lax
from jax.experimental import pallas as pl
from jax.experimental.pallas import tpu as pltpu
from jax.experimental.pallas import tpu_sc as plsc

EMB = 64
ROW = 128
LANES = 16
CHUNK = 64
NBUF = 8
AHEAD = NBUF - 2


@functools.lru_cache(maxsize=None)
def _build(n_rows: int, n_words: int):
    info = plsc.get_sparse_core_info()
    nc, ns = info.num_cores, info.num_subcores
    nw = nc * ns
    rpw = n_rows // nw
    chunks = rpw // CHUNK
    assert rpw % CHUNK == 0 and chunks % NBUF == 0
    outer = chunks // NBUF
    idx_rows = rpw // 128

    mesh = plsc.VectorSubcoreMesh(core_axis_name="c", subcore_axis_name="s")

    @functools.partial(
        pl.kernel,
        out_type=jax.ShapeDtypeStruct((n_rows, ROW), jnp.float32),
        mesh=mesh,
        scratch_types=[
            pltpu.VMEM((idx_rows, 128), jnp.int32),
            pltpu.VMEM((NBUF, CHUNK, ROW), jnp.float32),
            pltpu.VMEM((EMB,), jnp.float32),
            pltpu.SemaphoreType.DMA((NBUF,)),
            pltpu.SemaphoreType.DMA((NBUF,)),
        ],
    )
    def emb_kernel(tok_hbm, table_hbm, pos_hbm, out_hbm, idx_all, bufs, pos_v,
                   gsem, ssem):
        wid = lax.axis_index("s") * nc + lax.axis_index("c")
        base_row = wid * rpw
        base128 = wid * idx_rows

        pltpu.sync_copy(pos_hbm, pos_v)
        p = [pos_v[pl.ds(q * LANES, LANES)] for q in range(EMB // LANES)]
        pltpu.sync_copy(tok_hbm.at[pl.ds(base128, idx_rows)], idx_all)

        def idx_slice(c):
            if CHUNK == 128:
                return idx_all.at[c]
            per = 128 // CHUNK
            return idx_all.at[c // per, pl.ds((c % per) * CHUNK, CHUNK)]

        def gather(c, b):
            pltpu.async_copy(table_hbm.at[idx_slice(c)], bufs.at[b], gsem.at[b])

        def gather_wait(c, b):
            pltpu.make_async_copy(
                table_hbm.at[idx_slice(c)], bufs.at[b], gsem.at[b]).wait()

        def store(c, b):
            pltpu.async_copy(
                bufs.at[b], out_hbm.at[pl.ds(base_row + c * CHUNK, CHUNK)],
                ssem.at[b])

        def store_wait(b):
            pltpu.make_async_copy(
                bufs.at[b], out_hbm.at[pl.ds(base_row, CHUNK)],
                ssem.at[b]).wait()

        for f in range(AHEAD):
            gather(f, f)

        def outer_body(o, carry):
            for b in range(NBUF):
                t = o * NBUF + b
                f = t + AHEAD
                fb = (b + AHEAD) % NBUF

                @pl.when(t > 1)
                def _wait_store():
                    store_wait(fb)

                @pl.when(f < chunks)
                def _issue_gather():
                    gather(f, fb)

                gather_wait(t, b)

                def row_body(i, c):
                    for u in range(2):
                        for q in range(EMB // LANES):
                            sl = (b, i * 2 + u, pl.ds(q * LANES, LANES))
                            bufs[sl] = bufs[sl] + p[q]
                    return c

                lax.fori_loop(0, CHUNK // 2, row_body, 0)
                store(t, b)
            return carry

        lax.fori_loop(0, outer, outer_body, 0)

        for k in range(chunks - (NBUF - AHEAD), chunks):
            store_wait(k % NBUF)

    return emb_kernel


def kernel(tokens, table):
    t_dim, b_dim = tokens.shape
    n_rows = t_dim * b_dim
    n_words, emb = table.shape

    den = jnp.exp(-jnp.arange(0, emb, 2, dtype=jnp.float32) * math.log(10000.0) / emb)
    pos = jnp.zeros((emb,), dtype=jnp.float32)
    pos = pos.at[0::2].set(jnp.sin(t_dim * den))
    pos = pos.at[1::2].set(jnp.cos(t_dim * den))

    tok = tokens.reshape(n_rows // 128, 128).astype(jnp.int32)
    table_p = jnp.pad(table, ((0, 0), (0, ROW - emb)))
    out = _build(n_rows, n_words)(tok, table_p, pos)
    return out[:, :emb].reshape(t_dim, b_dim, emb)
